# baseline (speedup 1.0000x reference)
import functools

import jax
import jax.numpy as jnp
from jax import lax
from jax.experimental import pallas as pl
from jax.experimental.pallas import tpu as pltpu
from jax.experimental.pallas import tpu_sc as plsc

N, E, D = 10000, 320000, 128
DW = 144
NC, NS = 2, 16
NW = NC * NS
EPT = E // NW
CHUNK = 125
IB = 10
NBATCH = EPT // (CHUNK * IB)
NCHTOT = E // CHUNK
NPAD = 10000
ROWS_PER_SUB = NPAD // NS
ZR = 125
NB = 4


def _sc_aggregate(xb, edges):
    mesh = plsc.VectorSubcoreMesh(
        core_axis_name="c", subcore_axis_name="s", num_cores=NC, num_subcores=NS
    )

    @functools.partial(
        pl.kernel,
        out_type=jax.ShapeDtypeStruct((NC, NPAD, DW), jnp.bfloat16),
        mesh=mesh,
        scratch_types=[
            pltpu.VMEM((IB, CHUNK), jnp.int32),
            pltpu.VMEM((IB, CHUNK), jnp.int32),
            pltpu.VMEM((IB, CHUNK), jnp.int32),
            pltpu.VMEM((IB, CHUNK), jnp.int32),
            pltpu.VMEM((CHUNK, DW), jnp.bfloat16),
            pltpu.VMEM((CHUNK, DW), jnp.bfloat16),
            pltpu.VMEM((CHUNK, DW), jnp.bfloat16),
            pltpu.VMEM((CHUNK, DW), jnp.bfloat16),
            pltpu.VMEM_SHARED((NPAD, DW), jnp.bfloat16),
            pltpu.SemaphoreType.DMA,
            pltpu.SemaphoreType.DMA,
            pltpu.SemaphoreType.DMA,
            pltpu.SemaphoreType.DMA,
            pltpu.SemaphoreType.DMA,
            pltpu.SemaphoreType.DMA,
            pltpu.SemaphoreType.DMA,
            pltpu.SemaphoreType.DMA,
            pltpu.SemaphoreType.DMA,
            pltpu.SemaphoreType.DMA,
        ],
        compiler_params=pltpu.CompilerParams(use_tc_tiling_on_sc=False),
    )
    def k(xb_hbm, e_hbm, out_hbm, src0_v, dst0_v, src1_v, dst1_v,
          rows0, rows1, rows2, rows3, acc_sh,
          gsem0, gsem1, gsem2, gsem3, ssem0, ssem1, ssem2, ssem3,
          isem0, isem1):
        cid = lax.axis_index("c")
        sid = lax.axis_index("s")
        wid = cid * NS + sid

        zero32b = jnp.zeros((32,), jnp.bfloat16)

        @pl.loop(0, CHUNK)
        def _(r):
            @pl.loop(0, DW - 32, step=32)
            def _(cc):
                rows0[r, pl.ds(cc, 32)] = zero32b
            rows0[r, pl.ds(DW - 32, 32)] = zero32b

        @pl.loop(0, ROWS_PER_SUB // ZR)
        def _(kk):
            base = sid * ROWS_PER_SUB + kk * ZR
            pltpu.sync_copy(rows0.at[pl.ds(0, ZR)], acc_sh.at[pl.ds(base, ZR)])

        plsc.subcore_barrier()

        rows = (rows0, rows1, rows2, rows3)
        gsems = (gsem0, gsem1, gsem2, gsem3)
        ssems = (ssem0, ssem1, ssem2, ssem3)
        srcs = (src0_v, src1_v)
        dsts = (dst0_v, dst1_v)
        isems = (isem0, isem1)
        cbase = wid * (NBATCH * IB)
        last_cb = cbase + (NBATCH - 1) * IB

        pltpu.async_copy(e_hbm.at[0, pl.ds(cbase, IB)], src0_v, isem0)
        pltpu.async_copy(e_hbm.at[1, pl.ds(cbase, IB)], dst0_v, isem0)

        @pl.loop(0, NBATCH // 2)
        def _(go):
            for gg in range(2):
                sbuf, dbuf, isem = srcs[gg], dsts[gg], isems[gg]
                pltpu.make_async_copy(
                    e_hbm.at[0, pl.ds(cbase, IB)], sbuf, isem).wait()
                pltpu.make_async_copy(
                    e_hbm.at[1, pl.ds(cbase, IB)], dbuf, isem).wait()
                nxt = jnp.minimum(cbase + (2 * go + gg + 1) * IB, last_cb)
                ngg = (gg + 1) % 2
                pltpu.async_copy(e_hbm.at[0, pl.ds(nxt, IB)],
                                 srcs[ngg], isems[ngg])
                pltpu.async_copy(e_hbm.at[1, pl.ds(nxt, IB)],
                                 dsts[ngg], isems[ngg])

                gat = [None] * NB
                scats = [None] * NB
                for p in range(NB - 1):
                    gat[p] = pltpu.async_copy(
                        xb_hbm.at[sbuf.at[p]], rows[p], gsems[p])
                for c in range(IB):
                    b = c % NB
                    gat[b].wait()
                    if c + NB - 1 < IB:
                        nb = (c + NB - 1) % NB
                        if scats[nb] is not None:
                            scats[nb].wait()
                        gat[nb] = pltpu.async_copy(
                            xb_hbm.at[sbuf.at[c + NB - 1]], rows[nb], gsems[nb])
                    scats[b] = pltpu.async_copy(
                        rows[b], acc_sh.at[dbuf.at[c]], ssems[b], add=True)
                for sc in scats:
                    if sc is not None:
                        sc.wait()

        pltpu.make_async_copy(e_hbm.at[0, pl.ds(cbase, IB)], src0_v, isem0).wait()
        pltpu.make_async_copy(e_hbm.at[1, pl.ds(cbase, IB)], dst0_v, isem0).wait()

        base = sid * ROWS_PER_SUB
        pltpu.sync_copy(acc_sh.at[pl.ds(base, ROWS_PER_SUB)],
                        out_hbm.at[cid, pl.ds(base, ROWS_PER_SUB)])

    return k(xb, edges)


BLK = 2000


def _tc_combine(x, part, W_self, W_neigh, b, gamma, beta):
    def body(x_ref, p_ref, ws_ref, wn_ref, b_ref, g_ref, be_ref, o_ref):
        xb = x_ref[...]
        psum = p_ref[0].astype(jnp.float32) + p_ref[1].astype(jnp.float32)
        deg = psum[:, 128:129]
        agg = psum[:, :D] / jnp.maximum(deg, 1.0)
        h = jnp.dot(xb, ws_ref[...], preferred_element_type=jnp.float32)
        h = h + jnp.dot(agg, wn_ref[...], preferred_element_type=jnp.float32)
        h = h + b_ref[...]
        mu = jnp.mean(h, axis=1, keepdims=True)
        var = jnp.mean((h - mu) * (h - mu), axis=1, keepdims=True)
        h = (h - mu) * lax.rsqrt(var + 1e-5) * g_ref[...] + be_ref[...]
        o_ref[...] = jnp.maximum(h, 0.0) + xb

    return pl.pallas_call(
        body,
        grid=(N // BLK,),
        in_specs=[
            pl.BlockSpec((BLK, D), lambda i: (i, 0)),
            pl.BlockSpec((NC, BLK, DW), lambda i: (0, i, 0)),
            pl.BlockSpec((D, D), lambda i: (0, 0)),
            pl.BlockSpec((D, D), lambda i: (0, 0)),
            pl.BlockSpec((1, D), lambda i: (0, 0)),
            pl.BlockSpec((1, D), lambda i: (0, 0)),
            pl.BlockSpec((1, D), lambda i: (0, 0)),
        ],
        out_specs=pl.BlockSpec((BLK, D), lambda i: (i, 0)),
        out_shape=jax.ShapeDtypeStruct((N, D), jnp.float32),
    )(x, part, W_self, W_neigh, b, gamma, beta)


@jax.jit
def kernel(x, edge_index, W_self, W_neigh, b, gamma, beta):
    aug = jnp.concatenate(
        [x, jnp.ones((N, 1), jnp.float32), jnp.zeros((N, DW - D - 1), jnp.float32)],
        axis=1).astype(jnp.bfloat16)
    edges = edge_index.reshape(2, NCHTOT, CHUNK)
    part = _sc_aggregate(aug, edges)
    return _tc_combine(
        x, part, W_self, W_neigh,
        b.reshape(1, D), gamma.reshape(1, D), beta.reshape(1, D),
    )


# speedup vs baseline: 1.1978x; 1.1978x over previous
import functools

import jax
import jax.numpy as jnp
from jax import lax
from jax.experimental import pallas as pl
from jax.experimental.pallas import tpu as pltpu
from jax.experimental.pallas import tpu_sc as plsc

N, E, D = 10000, 320000, 128
HALF = D // 2
NC, NS = 2, 16
EPT = E // NS
CHUNK = 250
IB = 10
NBATCH = EPT // (CHUNK * IB)
NCHTOT = E // CHUNK
ZR = 125
NPAD = 10000
ROWS_PER_SUB = NPAD // NS
DEGW = 16
NB = 4


def _sc_aggregate(xs, edges):
    mesh = plsc.VectorSubcoreMesh(
        core_axis_name="c", subcore_axis_name="s", num_cores=NC, num_subcores=NS
    )

    @functools.partial(
        pl.kernel,
        out_type=[
            jax.ShapeDtypeStruct((NC, NPAD, HALF), jnp.bfloat16),
            jax.ShapeDtypeStruct((NC, NPAD, DEGW), jnp.float32),
        ],
        mesh=mesh,
        scratch_types=[
            pltpu.VMEM((IB, CHUNK), jnp.int32),
            pltpu.VMEM((IB, CHUNK), jnp.int32),
            pltpu.VMEM((IB, CHUNK), jnp.int32),
            pltpu.VMEM((IB, CHUNK), jnp.int32),
            pltpu.VMEM((CHUNK, HALF), jnp.bfloat16),
            pltpu.VMEM((CHUNK, HALF), jnp.bfloat16),
            pltpu.VMEM((CHUNK, HALF), jnp.bfloat16),
            pltpu.VMEM((CHUNK, HALF), jnp.bfloat16),
            pltpu.VMEM((CHUNK, DEGW), jnp.float32),
            pltpu.VMEM((CHUNK, DEGW), jnp.float32),
            pltpu.VMEM_SHARED((NPAD, HALF), jnp.bfloat16),
            pltpu.VMEM_SHARED((NPAD, DEGW), jnp.float32),
            pltpu.SemaphoreType.DMA,
            pltpu.SemaphoreType.DMA,
            pltpu.SemaphoreType.DMA,
            pltpu.SemaphoreType.DMA,
            pltpu.SemaphoreType.DMA,
            pltpu.SemaphoreType.DMA,
            pltpu.SemaphoreType.DMA,
            pltpu.SemaphoreType.DMA,
            pltpu.SemaphoreType.DMA,
            pltpu.SemaphoreType.DMA,
            pltpu.SemaphoreType.DMA,
        ],
        compiler_params=pltpu.CompilerParams(use_tc_tiling_on_sc=False),
    )
    def k(xs_hbm, e_hbm, out_hbm, deg_hbm, src0_v, dst0_v, src1_v, dst1_v,
          rows0, rows1, rows2, rows3, ones_v, zd_v, acc_sh, deg_sh,
          gsem0, gsem1, gsem2, gsem3, ssem0, ssem1, ssem2, ssem3,
          dsem, isem0, isem1):
        cid = lax.axis_index("c")
        sid = lax.axis_index("s")

        zero16 = jnp.zeros((16,), jnp.float32)
        one16 = jnp.ones((16,), jnp.float32)
        zero32b = jnp.zeros((32,), jnp.bfloat16)

        @pl.loop(0, CHUNK)
        def _(r):
            ones_v[r, :] = one16
            zd_v[r, :] = zero16

            @pl.loop(0, HALF, step=32)
            def _(cc):
                rows0[r, pl.ds(cc, 32)] = zero32b

        @pl.loop(0, ROWS_PER_SUB // ZR)
        def _(kk):
            base = sid * ROWS_PER_SUB + kk * ZR
            pltpu.sync_copy(rows0.at[pl.ds(0, ZR)], acc_sh.at[pl.ds(base, ZR)])
            pltpu.sync_copy(zd_v.at[pl.ds(0, ZR)], deg_sh.at[pl.ds(base, ZR)])

        plsc.subcore_barrier()

        xh = xs_hbm.at[cid]
        rows = (rows0, rows1, rows2, rows3)
        gsems = (gsem0, gsem1, gsem2, gsem3)
        ssems = (ssem0, ssem1, ssem2, ssem3)
        srcs = (src0_v, src1_v)
        dsts = (dst0_v, dst1_v)
        isems = (isem0, isem1)
        cbase = sid * (NBATCH * IB)
        last_cb = cbase + (NBATCH - 1) * IB

        pltpu.async_copy(e_hbm.at[0, pl.ds(cbase, IB)], src0_v, isem0)
        pltpu.async_copy(e_hbm.at[1, pl.ds(cbase, IB)], dst0_v, isem0)

        @pl.loop(0, NBATCH // 2)
        def _(go):
            for gg in range(2):
                sbuf, dbuf, isem = srcs[gg], dsts[gg], isems[gg]
                pltpu.make_async_copy(
                    e_hbm.at[0, pl.ds(cbase, IB)], sbuf, isem).wait()
                pltpu.make_async_copy(
                    e_hbm.at[1, pl.ds(cbase, IB)], dbuf, isem).wait()
                nxt = jnp.minimum(cbase + (2 * go + gg + 1) * IB, last_cb)
                ngg = (gg + 1) % 2
                pltpu.async_copy(e_hbm.at[0, pl.ds(nxt, IB)],
                                 srcs[ngg], isems[ngg])
                pltpu.async_copy(e_hbm.at[1, pl.ds(nxt, IB)],
                                 dsts[ngg], isems[ngg])

                gat = [None] * NB
                scats = [None] * NB
                deg_descs = [[], []]
                for p in range(NB - 1):
                    gat[p] = pltpu.async_copy(
                        xh.at[sbuf.at[p]], rows[p], gsems[p])
                for c in range(IB):
                    b = c % NB
                    gat[b].wait()
                    if c + NB - 1 < IB:
                        nb = (c + NB - 1) % NB
                        if scats[nb] is not None:
                            scats[nb].wait()
                        gat[nb] = pltpu.async_copy(
                            xh.at[sbuf.at[c + NB - 1]], rows[nb], gsems[nb])
                    scats[b] = pltpu.async_copy(
                        rows[b], acc_sh.at[dbuf.at[c]], ssems[b], add=True)
                    par = c % 2

                    @pl.when(cid == par)
                    def _():
                        deg_descs[par].append(pltpu.async_copy(
                            ones_v, deg_sh.at[dbuf.at[c]], dsem, add=True))
                for sc in scats:
                    if sc is not None:
                        sc.wait()
                for par in range(2):
                    @pl.when(cid == par)
                    def _():
                        for dd in deg_descs[par]:
                            dd.wait()

        pltpu.make_async_copy(e_hbm.at[0, pl.ds(cbase, IB)], src0_v, isem0).wait()
        pltpu.make_async_copy(e_hbm.at[1, pl.ds(cbase, IB)], dst0_v, isem0).wait()

        base = sid * ROWS_PER_SUB
        pltpu.sync_copy(acc_sh.at[pl.ds(base, ROWS_PER_SUB)],
                        out_hbm.at[cid, pl.ds(base, ROWS_PER_SUB)])
        pltpu.sync_copy(deg_sh.at[pl.ds(base, ROWS_PER_SUB)],
                        deg_hbm.at[cid, pl.ds(base, ROWS_PER_SUB)])

    return k(xs, edges)


BLK = 2000


def _tc_combine(x, part, degp, W_self, W_neigh, b, gamma, beta):
    def body(x_ref, p_ref, d_ref, ws_ref, wn_ref, b_ref, g_ref, be_ref, o_ref):
        xb = x_ref[...]
        psum = jnp.concatenate([p_ref[0], p_ref[1]], axis=1).astype(jnp.float32)
        deg = d_ref[0, :, 0:1] + d_ref[1, :, 0:1]
        agg = psum / jnp.maximum(deg, 1.0)
        h = jnp.dot(xb, ws_ref[...], preferred_element_type=jnp.float32)
        h = h + jnp.dot(agg, wn_ref[...], preferred_element_type=jnp.float32)
        h = h + b_ref[...]
        mu = jnp.mean(h, axis=1, keepdims=True)
        var = jnp.mean((h - mu) * (h - mu), axis=1, keepdims=True)
        h = (h - mu) * lax.rsqrt(var + 1e-5) * g_ref[...] + be_ref[...]
        o_ref[...] = jnp.maximum(h, 0.0) + xb

    return pl.pallas_call(
        body,
        grid=(N // BLK,),
        in_specs=[
            pl.BlockSpec((BLK, D), lambda i: (i, 0)),
            pl.BlockSpec((NC, BLK, HALF), lambda i: (0, i, 0)),
            pl.BlockSpec((NC, BLK, DEGW), lambda i: (0, i, 0)),
            pl.BlockSpec((D, D), lambda i: (0, 0)),
            pl.BlockSpec((D, D), lambda i: (0, 0)),
            pl.BlockSpec((1, D), lambda i: (0, 0)),
            pl.BlockSpec((1, D), lambda i: (0, 0)),
            pl.BlockSpec((1, D), lambda i: (0, 0)),
        ],
        out_specs=pl.BlockSpec((BLK, D), lambda i: (i, 0)),
        out_shape=jax.ShapeDtypeStruct((N, D), jnp.float32),
    )(x, part, degp, W_self, W_neigh, b, gamma, beta)


@jax.jit
def kernel(x, edge_index, W_self, W_neigh, b, gamma, beta):
    xs = jnp.stack([x[:, :HALF], x[:, HALF:]]).astype(jnp.bfloat16)
    edges = edge_index.reshape(2, NCHTOT, CHUNK)
    part, degp = _sc_aggregate(xs, edges)
    return _tc_combine(
        x, part, degp, W_self, W_neigh,
        b.reshape(1, D), gamma.reshape(1, D), beta.reshape(1, D),
    )
